# baseline (speedup 1.0000x reference)
import functools

import jax
import jax.numpy as jnp
from jax import lax
from jax.experimental import pallas as pl
from jax.experimental.pallas import tpu as pltpu
from jax.experimental.pallas import tpu_sc as plsc

_D = 64
_DPAD = 128
_NW = 32
_CHUNK = 640
_SLAB = 128
_NSLAB = _CHUNK // _SLAB


def _make_kernel(n_rows):
    rows_per_w = n_rows // _NW
    n_chunks = rows_per_w // _CHUNK
    assert n_chunks % 2 == 0
    mesh = plsc.VectorSubcoreMesh(core_axis_name="c", subcore_axis_name="s")

    @functools.partial(
        pl.kernel,
        mesh=mesh,
        out_type=jax.ShapeDtypeStruct((n_rows, _DPAD), jnp.float32),
        scratch_types=[
            pltpu.VMEM((_CHUNK,), jnp.int32),
            pltpu.VMEM((_CHUNK,), jnp.int32),
            pltpu.VMEM((_CHUNK, _D), jnp.float32),
            pltpu.VMEM((_CHUNK, _D), jnp.float32),
            pltpu.VMEM((4, _D), jnp.float32),
            pltpu.SemaphoreType.DMA,
            pltpu.SemaphoreType.DMA,
            pltpu.SemaphoreType.DMA,
            pltpu.SemaphoreType.DMA,
        ],
        compiler_params=pltpu.CompilerParams(use_tc_tiling_on_sc=False),
    )
    def k(ids_hbm, table_hbm, spec_hbm, out_hbm, idx0, idx1, rows0, rows1,
          spec_v, gs0, gs1, os0, os1):
        idx = (idx0, idx1)
        rows = (rows0, rows1)
        gsem = (gs0, gs1)
        osem = (os0, os1)
        wid = lax.axis_index("s") * 2 + lax.axis_index("c")
        base = wid * rows_per_w
        pltpu.sync_copy(spec_hbm, spec_v)

        def load_and_fire(g, b):
            cb = base + g * _CHUNK
            pltpu.sync_copy(ids_hbm.at[pl.ds(cb, _CHUNK)], idx[b])
            for j in range(_NSLAB):
                pltpu.async_copy(
                    table_hbm.at[idx[b].at[pl.ds(j * _SLAB, _SLAB)]],
                    rows[b].at[pl.ds(j * _SLAB, _SLAB)],
                    gsem[b],
                )

        def drain_gather(b):
            for j in range(_NSLAB):
                pltpu.make_async_copy(
                    table_hbm.at[idx[b].at[pl.ds(j * _SLAB, _SLAB)]],
                    rows[b].at[pl.ds(j * _SLAB, _SLAB)],
                    gsem[b],
                ).wait()

        def fire_writeback(g, b):
            cb = base + g * _CHUNK
            pltpu.async_copy(
                rows[b], out_hbm.at[pl.ds(cb, _CHUNK), pl.ds(0, _D)],
                osem[b])

        def drain_writeback(b):
            pltpu.make_async_copy(
                rows[b], out_hbm.at[pl.ds(base, _CHUNK), pl.ds(0, _D)],
                osem[b]).wait()

        def detect(b):
            def det(t, a):
                ids = idx[b][pl.ds(t * 16, 16)]
                return a | jnp.where(ids < 3, 1, 0)

            acc = lax.fori_loop(0, _CHUNK // 16, det,
                                jnp.zeros((16,), jnp.int32))
            flag = acc[0]
            for i in range(1, 16):
                flag = flag | acc[i]
            return flag

        def patch(b, flag):
            @pl.when(flag > 0)
            def _fix():
                def fix_group(t, carry1):
                    ids = idx[b][pl.ds(t * 16, 16)]
                    for lane in range(16):
                        id_s = ids[lane]

                        @pl.when(id_s < 3)
                        def _one():
                            p = t * 16 + lane
                            for q in range(_D // 16):
                                rows[b][p, pl.ds(q * 16, 16)] = (
                                    spec_v[id_s, pl.ds(q * 16, 16)])

                    return carry1

                lax.fori_loop(0, _CHUNK // 16, fix_group, 0)

        load_and_fire(0, 0)

        def pair_body(p, carry0):
            for b in (0, 1):
                g = 2 * p + b

                @pl.when(g + 1 < n_chunks)
                def _prefetch():
                    @pl.when(g >= 1)
                    def _reuse():
                        drain_writeback(1 - b)

                    load_and_fire(g + 1, 1 - b)

                flag = detect(b)
                drain_gather(b)
                patch(b, flag)
                fire_writeback(g, b)
            return carry0

        lax.fori_loop(0, n_chunks // 2, pair_body, 0)
        drain_writeback(0)
        drain_writeback(1)

    return k


def kernel(road_ids, road_geo_data, table, unk_emb, pad_emb, mask_emb):
    del road_geo_data
    n_rows = road_ids.shape[0] * road_ids.shape[1]
    ids_flat = road_ids.reshape(n_rows)
    t2 = lax.optimization_barrier(table.reshape(table.shape[0] // 2, 2 * _D))
    tpad = t2.reshape(table.shape)
    spec = jnp.stack(
        [pad_emb, mask_emb, unk_emb, jnp.zeros_like(pad_emb)], axis=0)
    outp = _make_kernel(n_rows)(ids_flat, tpad, spec)
    return outp[:, :_D].reshape(road_ids.shape[0], road_ids.shape[1], _D)


# speedup vs baseline: 1.0051x; 1.0051x over previous
import functools

import jax
import jax.numpy as jnp
from jax import lax
from jax.experimental import pallas as pl
from jax.experimental.pallas import tpu as pltpu
from jax.experimental.pallas import tpu_sc as plsc

_D = 64
_DPAD = 128
_NW = 32
_CHUNK = 640
_SLAB = 128
_NSLAB = _CHUNK // _SLAB


def _make_kernel(n_rows):
    rows_per_w = n_rows // _NW
    n_chunks = rows_per_w // _CHUNK
    assert n_chunks % 2 == 0
    mesh = plsc.VectorSubcoreMesh(core_axis_name="c", subcore_axis_name="s")

    @functools.partial(
        pl.kernel,
        mesh=mesh,
        out_type=jax.ShapeDtypeStruct((n_rows, _DPAD), jnp.float32),
        scratch_types=[
            pltpu.VMEM((_CHUNK,), jnp.int32),
            pltpu.VMEM((_CHUNK,), jnp.int32),
            pltpu.VMEM((_CHUNK, _D), jnp.float32),
            pltpu.VMEM((_CHUNK, _D), jnp.float32),
            pltpu.VMEM((4, _D), jnp.float32),
            pltpu.SemaphoreType.DMA,
            pltpu.SemaphoreType.DMA,
            pltpu.SemaphoreType.DMA,
            pltpu.SemaphoreType.DMA,
        ],
        compiler_params=pltpu.CompilerParams(use_tc_tiling_on_sc=False),
    )
    def k(ids_hbm, table_hbm, spec_hbm, out_hbm, idx0, idx1, rows0, rows1,
          spec_v, gs0, gs1, os0, os1):
        idx = (idx0, idx1)
        rows = (rows0, rows1)
        gsem = (gs0, gs1)
        osem = (os0, os1)
        wid = lax.axis_index("s") * 2 + lax.axis_index("c")
        base = wid * rows_per_w
        pltpu.sync_copy(spec_hbm, spec_v)

        def load_and_fire(g, b):
            cb = base + g * _CHUNK
            pltpu.sync_copy(ids_hbm.at[pl.ds(cb, _CHUNK)], idx[b])
            for j in range(_NSLAB):
                pltpu.async_copy(
                    table_hbm.at[idx[b].at[pl.ds(j * _SLAB, _SLAB)]],
                    rows[b].at[pl.ds(j * _SLAB, _SLAB)],
                    gsem[b],
                )

        def drain_gather(b):
            for j in range(_NSLAB):
                pltpu.make_async_copy(
                    table_hbm.at[idx[b].at[pl.ds(j * _SLAB, _SLAB)]],
                    rows[b].at[pl.ds(j * _SLAB, _SLAB)],
                    gsem[b],
                ).wait()

        def fire_writeback(g, b):
            cb = base + g * _CHUNK
            pltpu.async_copy(
                rows[b], out_hbm.at[pl.ds(cb, _CHUNK), pl.ds(0, _D)],
                osem[b])

        def drain_writeback(b):
            pltpu.make_async_copy(
                rows[b], out_hbm.at[pl.ds(base, _CHUNK), pl.ds(0, _D)],
                osem[b]).wait()

        def detect(b):
            def det(t, a):
                ids = idx[b][pl.ds(t * 16, 16)]
                return a | jnp.where(ids < 3, 1, 0)

            acc = lax.fori_loop(0, _CHUNK // 16, det,
                                jnp.zeros((16,), jnp.int32))
            flag = acc[0]
            for i in range(1, 16):
                flag = flag | acc[i]
            return flag

        def patch(b, flag):
            @pl.when(flag > 0)
            def _fix():
                def fix_group(t, carry1):
                    ids = idx[b][pl.ds(t * 16, 16)]
                    for lane in range(16):
                        id_s = ids[lane]

                        @pl.when(id_s < 3)
                        def _one():
                            p = t * 16 + lane
                            for q in range(_D // 16):
                                rows[b][p, pl.ds(q * 16, 16)] = (
                                    spec_v[id_s, pl.ds(q * 16, 16)])

                    return carry1

                lax.fori_loop(0, _CHUNK // 16, fix_group, 0)

        load_and_fire(0, 0)

        def pair_body(p, carry0):
            for b in (0, 1):
                g = 2 * p + b

                @pl.when(g + 1 < n_chunks)
                def _prefetch():
                    @pl.when(g >= 1)
                    def _reuse():
                        drain_writeback(1 - b)

                    load_and_fire(g + 1, 1 - b)

                flag = detect(b)
                drain_gather(b)
                patch(b, flag)
                fire_writeback(g, b)
            return carry0

        lax.fori_loop(0, n_chunks // 2, pair_body, 0)
        drain_writeback(0)
        drain_writeback(1)

    return k


def kernel(road_ids, road_geo_data, table, unk_emb, pad_emb, mask_emb):
    del road_geo_data
    n_rows = road_ids.shape[0] * road_ids.shape[1]
    ids_flat = road_ids.reshape(n_rows)
    tpad = table
    spec = jnp.stack(
        [pad_emb, mask_emb, unk_emb, jnp.zeros_like(pad_emb)], axis=0)
    outp = _make_kernel(n_rows)(ids_flat, tpad, spec)
    return outp[:, :_D].reshape(road_ids.shape[0], road_ids.shape[1], _D)
